# baseline (speedup 1.0000x reference)
import functools

import jax
import jax.numpy as jnp
from jax import lax
from jax.experimental import pallas as pl
from jax.experimental.pallas import tpu as pltpu
from jax.experimental.pallas import tpu_sc as plsc

L = 16
NC, NS = 2, 16
NW = NC * NS
R_CHUNK = 32
IDXROW = 64


def _bcast_lane(v, k):
    return lax.gather(
        v, jnp.full((L, 1), k, jnp.int32),
        lax.GatherDimensionNumbers(
            offset_dims=(), collapsed_slice_dims=(0,), start_index_map=(0,)),
        slice_sizes=(1,),
        mode=lax.GatherScatterMode.PROMISE_IN_BOUNDS)


@functools.lru_cache(maxsize=None)
def _build(batch, fields, vocab, dim):
    look = R_CHUNK * fields
    per_w_rows = batch // NW
    n_chunks = per_w_rows // R_CHUNK
    kstream = look // IDXROW
    n_grp = look // L
    max_idx = vocab - 1

    mesh = plsc.VectorSubcoreMesh(core_axis_name="c", subcore_axis_name="s")

    @functools.partial(
        pl.kernel,
        out_type=jax.ShapeDtypeStruct((batch, fields, dim), jnp.float32),
        mesh=mesh,
        compiler_params=pltpu.CompilerParams(use_tc_tiling_on_sc=False),
        scratch_types=[
            pltpu.VMEM((look,), jnp.float32),
            pltpu.VMEM((kstream, IDXROW), jnp.int32),
            pltpu.VMEM((kstream, IDXROW), jnp.int32),
            pltpu.VMEM((look, dim), jnp.float32),
            pltpu.VMEM((look, dim), jnp.float32),
            pltpu.VMEM((R_CHUNK, fields, dim), jnp.float32),
            pltpu.SemaphoreType.DMA,
        ],
    )
    def impl(cont_hbm, w_hbm, out_hbm, cont_v, idx_lo_v, idx_hi_v,
             lo_v, hi_v, out_v, sem):
        wid = lax.axis_index("s") * NC + lax.axis_index("c")

        def chunk_body(g, _):
            row0 = wid * per_w_rows + g * R_CHUNK
            base = row0 * fields
            pltpu.sync_copy(cont_hbm.at[pl.ds(base, look)], cont_v)

            def idx_body(t, _):
                x = cont_v[pl.ds(t * L, L)]
                il = x.astype(jnp.int32)
                w = x - il.astype(jnp.float32)
                ih = jnp.minimum(il + 1, max_idx)
                r = t // (IDXROW // L)
                c = (t % (IDXROW // L)) * L
                idx_lo_v[r, pl.ds(c, L)] = il
                idx_hi_v[r, pl.ds(c, L)] = ih
                cont_v[pl.ds(t * L, L)] = w
                return 0

            lax.fori_loop(0, n_grp, idx_body, 0)

            copies = []
            for j in range(kstream):
                copies.append(pltpu.async_copy(
                    w_hbm.at[idx_lo_v.at[j]],
                    lo_v.at[pl.ds(j * IDXROW, IDXROW)], sem))
                copies.append(pltpu.async_copy(
                    w_hbm.at[idx_hi_v.at[j]],
                    hi_v.at[pl.ds(j * IDXROW, IDXROW)], sem))
            for cp in copies:
                cp.wait()

            def blend_body(t, _):
                w16 = cont_v[pl.ds(t * L, L)]
                for k in range(L):
                    i = t * L + k
                    r = i // fields
                    f = i - r * fields
                    wv = _bcast_lane(w16, k)
                    for d in range(dim // L):
                        lo = lo_v[i, pl.ds(d * L, L)]
                        hi = hi_v[i, pl.ds(d * L, L)]
                        out_v[r, f, pl.ds(d * L, L)] = lo + wv * (hi - lo)
                return 0

            lax.fori_loop(0, n_grp, blend_body, 0)

            pltpu.sync_copy(out_v, out_hbm.at[pl.ds(row0, R_CHUNK)])
            return 0

        lax.fori_loop(0, n_chunks, chunk_body, 0)

    return impl


def kernel(continuous_idx, W):
    batch, fields = continuous_idx.shape
    vocab, dim = W.shape
    impl = _build(batch, fields, vocab, dim)
    return impl(continuous_idx.reshape(batch * fields), W)


# speedup vs baseline: 1.1931x; 1.1931x over previous
import functools

import jax
import jax.numpy as jnp
from jax import lax
from jax.experimental import pallas as pl
from jax.experimental.pallas import tpu as pltpu
from jax.experimental.pallas import tpu_sc as plsc

L = 16
NC, NS = 2, 16
NW = NC * NS
CHUNK = 1024
IDXROW = 128
KSTREAM = CHUNK // IDXROW
OUTW = 128


def _bcast_lane(v, k):
    return lax.gather(
        v, jnp.full((L, 1), k, jnp.int32),
        lax.GatherDimensionNumbers(
            offset_dims=(), collapsed_slice_dims=(0,), start_index_map=(0,)),
        slice_sizes=(1,),
        mode=lax.GatherScatterMode.PROMISE_IN_BOUNDS)


@functools.lru_cache(maxsize=None)
def _build(n_total, vocab, dim):
    per_w = n_total // NW
    n_chunks = per_w // CHUNK
    pack = OUTW // dim
    out_rows = n_total // pack
    orows_chunk = CHUNK // pack
    max_idx = vocab - 1

    mesh = plsc.VectorSubcoreMesh(core_axis_name="c", subcore_axis_name="s")

    @functools.partial(
        pl.kernel,
        out_type=jax.ShapeDtypeStruct((out_rows, OUTW), jnp.float32),
        mesh=mesh,
        compiler_params=pltpu.CompilerParams(use_tc_tiling_on_sc=False),
        scratch_types=[
            pltpu.VMEM((CHUNK,), jnp.float32),
            pltpu.VMEM((KSTREAM, IDXROW), jnp.int32),
            pltpu.VMEM((KSTREAM, IDXROW), jnp.int32),
            pltpu.VMEM((CHUNK, dim), jnp.float32),
            pltpu.VMEM((CHUNK, dim), jnp.float32),
            pltpu.VMEM((orows_chunk, OUTW), jnp.float32),
            pltpu.SemaphoreType.DMA,
        ],
    )
    def impl(cont_hbm, w_hbm, out_hbm, cont_v, idx_lo_v, idx_hi_v,
             lo_v, hi_v, out_v, sem):
        wid = lax.axis_index("s") * NC + lax.axis_index("c")

        def chunk_body(g, _):
            base = wid * per_w + g * CHUNK
            pltpu.sync_copy(cont_hbm.at[pl.ds(base, CHUNK)], cont_v)

            def idx_body(t, _):
                x = cont_v[pl.ds(t * L, L)]
                il = x.astype(jnp.int32)
                w = x - il.astype(jnp.float32)
                ih = jnp.minimum(il + 1, max_idx)
                r = t // (IDXROW // L)
                c = (t % (IDXROW // L)) * L
                idx_lo_v[r, pl.ds(c, L)] = il
                idx_hi_v[r, pl.ds(c, L)] = ih
                cont_v[pl.ds(t * L, L)] = w
                return 0

            lax.fori_loop(0, CHUNK // L, idx_body, 0)

            copies = []
            for j in range(KSTREAM):
                copies.append(pltpu.async_copy(
                    w_hbm.at[idx_lo_v.at[j]],
                    lo_v.at[pl.ds(j * IDXROW, IDXROW)], sem))
                copies.append(pltpu.async_copy(
                    w_hbm.at[idx_hi_v.at[j]],
                    hi_v.at[pl.ds(j * IDXROW, IDXROW)], sem))
            for cp in copies:
                cp.wait()

            def blend_body(t, _):
                w16 = cont_v[pl.ds(t * L, L)]
                for k in range(L):
                    i = t * L + k
                    orow = t * (L // pack) + k // pack
                    ocol = (k % pack) * dim
                    wv = _bcast_lane(w16, k)
                    for d in range(dim // L):
                        lo = lo_v[i, pl.ds(d * L, L)]
                        hi = hi_v[i, pl.ds(d * L, L)]
                        out_v[orow, pl.ds(ocol + d * L, L)] = lo + wv * (hi - lo)
                return 0

            lax.fori_loop(0, CHUNK // L, blend_body, 0)

            pltpu.sync_copy(out_v, out_hbm.at[pl.ds(base // pack, orows_chunk)])
            return 0

        lax.fori_loop(0, n_chunks, chunk_body, 0)

    return impl


def kernel(continuous_idx, W):
    batch, fields = continuous_idx.shape
    vocab, dim = W.shape
    n_total = batch * fields
    impl = _build(n_total, vocab, dim)
    out = impl(continuous_idx.reshape(n_total), W)
    return out.reshape(batch, fields, dim)
